# speedup vs baseline: 1.0123x
import functools

import jax
import jax.numpy as jnp
from jax.experimental import pallas as pl
from jax.experimental.pallas import tpu as pltpu

N = 10000
E = 320000
D = 128
L = 4

EB = 1600
G = E // EB


def _edge_body(ee_ref, dhs_ref, ehs_ref, bhs_ref, wc_ref, bc_ref,
               enew_ref, sig_ref, msg_ref, stat_ref):
    ee = ee_ref[...]
    ce = jnp.dot(ee, wc_ref[...], preferred_element_type=jnp.float32) + bc_ref[...]
    enew = dhs_ref[...] + ehs_ref[...] + ce
    sig = jax.nn.sigmoid(enew)
    enew_ref[...] = enew
    sig_ref[...] = sig
    msg_ref[...] = bhs_ref[...] * sig
    stat_ref[0, 0, :] = jnp.sum(enew, axis=0)
    stat_ref[0, 1, :] = jnp.sum(enew * enew, axis=0)


def _edge_stage(ee, dhs, ehs, bhs, wc, bc):
    out_shapes = (
        jax.ShapeDtypeStruct((E, D), jnp.float32),
        jax.ShapeDtypeStruct((E, D), jnp.float32),
        jax.ShapeDtypeStruct((E, D), jnp.float32),
        jax.ShapeDtypeStruct((G, 2, D), jnp.float32),
    )
    eb_spec = pl.BlockSpec((EB, D), lambda i: (i, 0))
    return pl.pallas_call(
        _edge_body,
        grid=(G,),
        in_specs=[
            eb_spec, eb_spec, eb_spec, eb_spec,
            pl.BlockSpec((D, D), lambda i: (0, 0)),
            pl.BlockSpec((D,), lambda i: (0,)),
        ],
        out_specs=(
            eb_spec, eb_spec, eb_spec,
            pl.BlockSpec((1, 2, D), lambda i: (i, 0, 0)),
        ),
        out_shape=out_shapes,
    )(ee, dhs, ehs, bhs, wc, bc)


def kernel(h, e, edge_index, p, snorm_n, atom_emb, bond_emb, Wp, bp,
           Wa, ba, Wb, bb, Wc, bc, Wd, bd, We, be,
           bn_gh, bn_bh, bn_ge, bn_be, Wm0, bm0, Wm1, bm1, Wm2, bm2):
    hh = atom_emb[h] + p @ Wp + bp
    ee = bond_emb[e]
    src = edge_index[0]
    dst = edge_index[1]
    for l in range(L):
        h_in = hh
        e_in = ee
        Ah = hh @ Wa[l] + ba[l]
        Bh = hh @ Wb[l] + bb[l]
        Dh = hh @ Wd[l] + bd[l]
        Eh = hh @ We[l] + be[l]
        e_new, sigma, msg, stats = _edge_stage(ee, Dh[src], Eh[dst], Bh[src],
                                               Wc[l], bc[l])
        num = jax.ops.segment_sum(msg, dst, num_segments=N)
        den = jax.ops.segment_sum(sigma, dst, num_segments=N)
        h_new = Ah + num / (den + 1e-6)
        mu_h = h_new.mean(axis=0)
        var_h = h_new.var(axis=0)
        h_new = (h_new - mu_h) / jnp.sqrt(var_h + 1e-5) * bn_gh[l] + bn_bh[l]
        s = stats.sum(axis=0)
        mu_e = s[0] / E
        var_e = s[1] / E - mu_e * mu_e
        e_bn = (e_new - mu_e) / jnp.sqrt(var_e + 1e-5) * bn_ge[l] + bn_be[l]
        hh = jax.nn.relu(h_new) + h_in
        ee = jax.nn.relu(e_bn) + e_in
    hg = hh.mean(axis=0)
    y = jax.nn.relu(hg @ Wm0 + bm0)
    y = jax.nn.relu(y @ Wm1 + bm1)
    y = y @ Wm2 + bm2
    return y


# speedup vs baseline: 1.0860x; 1.0860x over previous
import functools

import jax
import jax.numpy as jnp
from jax import lax
from jax.experimental import pallas as pl
from jax.experimental.pallas import tpu as pltpu
from jax.experimental.pallas import tpu_sc as plsc

N = 10000
E = 320000
EH = E // 2
D = 128
L = 4
H = 64
NPAD = 10240
NC = 2
NS = 16
RPT = EH // NS
K = 80
CH = RPT // K
NPT = NPAD // NS

RNG = 5056
ACC = 5120
APT = ACC // NS

EBH = 1600
G2 = EH // EBH


def _sc_edge_body(*refs):
    (src_hbm, dst_hbm, tsb, teh, ce_hbm,
     enew_hbm, nd_hbm, stats_hbm,
     s0v, s1v, d0v, d1v, d0x, d1x,
     bufT0, bufT1, bufE0, bufE1, bufC, bufM0, bufM1,
     statb, acc) = refs
    c = lax.axis_index("c")
    s = lax.axis_index("s")
    wid = c * NS + s

    zero = jnp.zeros((16,), jnp.float32)

    def _zero_acc():
        def _zrow(r, _):
            for j in range(D // 16):
                bufM0[r, pl.ds(j * 16, 16)] = zero
            return 0
        lax.fori_loop(0, K, _zrow, 0)
        for k in range(APT // K):
            pltpu.sync_copy(bufM0, acc.at[pl.ds(s * APT + k * K, K)])
        plsc.subcore_barrier()

    def _clamp(dv, xv, lo):
        for j in range(K // 16):
            sl = pl.ds(j * 16, 16)
            d = dv[sl] - lo
            ok = (d >= 0) & (d < RNG)
            xv[sl] = jnp.where(ok, d, RNG)

    def _writeout(p):
        plsc.subcore_barrier()
        for k in range(APT // K):
            nsl = pl.ds(s * APT + k * K, K)
            osl = pl.ds(c * (2 * ACC) + p * ACC + s * APT + k * K, K)
            pltpu.sync_copy(acc.at[nsl], nd_hbm.at[osl])
        plsc.subcore_barrier()

    _zero_acc()

    def _chunk(i, carry):
        base = s * RPT + i * K
        pltpu.sync_copy(src_hbm.at[pl.ds(base, K)], s0v)
        pltpu.sync_copy(src_hbm.at[pl.ds(EH + base, K)], s1v)
        pltpu.sync_copy(dst_hbm.at[pl.ds(base, K)], d0v)
        pltpu.sync_copy(dst_hbm.at[pl.ds(EH + base, K)], d1v)
        coff = c * N
        for j in range(K // 16):
            sl = pl.ds(j * 16, 16)
            s0v[sl] = s0v[sl] + coff
            s1v[sl] = s1v[sl] + coff
        pltpu.sync_copy(tsb.at[s0v], bufT0)
        pltpu.sync_copy(tsb.at[s1v], bufT1)
        pltpu.sync_copy(teh.at[d0v], bufE0)
        pltpu.sync_copy(teh.at[d1v], bufE1)
        pltpu.sync_copy(ce_hbm.at[pl.ds(c * EH + base, K)], bufC)

        def _row(q, cr):
            acc_c = list(cr)
            for half in range(2):
                bt = bufT0 if half == 0 else bufT1
                be = bufE0 if half == 0 else bufE1
                bm = bufM0 if half == 0 else bufM1
                for j in range(H // 16):
                    slj = pl.ds(j * 16, 16)
                    slc = pl.ds(half * H + j * 16, 16)
                    sle = pl.ds(c * H + j * 16, 16)
                    slb = pl.ds(H + j * 16, 16)
                    x = bt[q, slj] + be[q, sle] + bufC[q, slc]
                    sg = 1.0 / (1.0 + jnp.exp(-x))
                    bufC[q, slc] = x
                    bm[q, slj] = bt[q, slb] * sg
                    bm[q, slb] = sg
                    acc_c[j] = acc_c[j] + x
                    acc_c[4 + j] = acc_c[4 + j] + x * x
            return tuple(acc_c)

        carry = lax.fori_loop(0, K, _row, carry)
        pltpu.sync_copy(bufC, enew_hbm.at[pl.ds(c * EH + base, K)])
        _clamp(d0v, d0x, 0)
        _clamp(d1v, d1x, 0)
        pltpu.sync_copy(bufM0, acc.at[d0x], add=True)
        pltpu.sync_copy(bufM1, acc.at[d1x], add=True)
        return carry

    init = tuple(zero for _ in range(8))
    carry = lax.fori_loop(0, CH, _chunk, init)

    for j in range(H // 16):
        statb[0, pl.ds(j * 16, 16)] = carry[j]
        statb[0, pl.ds(H + j * 16, 16)] = carry[4 + j]
    pltpu.sync_copy(statb, stats_hbm.at[pl.ds(wid, 1)])

    _writeout(0)

    _zero_acc()

    def _chunk2(i, _):
        base = s * RPT + i * K
        pltpu.sync_copy(src_hbm.at[pl.ds(base, K)], s0v)
        pltpu.sync_copy(src_hbm.at[pl.ds(EH + base, K)], s1v)
        pltpu.sync_copy(dst_hbm.at[pl.ds(base, K)], d0v)
        pltpu.sync_copy(dst_hbm.at[pl.ds(EH + base, K)], d1v)
        coff = c * N
        for j in range(K // 16):
            sl = pl.ds(j * 16, 16)
            s0v[sl] = s0v[sl] + coff
            s1v[sl] = s1v[sl] + coff
        pltpu.sync_copy(tsb.at[s0v], bufT0)
        pltpu.sync_copy(tsb.at[s1v], bufT1)
        pltpu.sync_copy(enew_hbm.at[pl.ds(c * EH + base, K)], bufC)

        def _row(q, _):
            for half in range(2):
                bt = bufT0 if half == 0 else bufT1
                bm = bufM0 if half == 0 else bufM1
                for j in range(H // 16):
                    slj = pl.ds(j * 16, 16)
                    slc = pl.ds(half * H + j * 16, 16)
                    slb = pl.ds(H + j * 16, 16)
                    x = bufC[q, slc]
                    sg = 1.0 / (1.0 + jnp.exp(-x))
                    bm[q, slj] = bt[q, slb] * sg
                    bm[q, slb] = sg
            return 0

        lax.fori_loop(0, K, _row, 0)
        _clamp(d0v, d0x, RNG)
        _clamp(d1v, d1x, RNG)
        pltpu.sync_copy(bufM0, acc.at[d0x], add=True)
        pltpu.sync_copy(bufM1, acc.at[d1x], add=True)
        return 0

    lax.fori_loop(0, CH, _chunk2, 0)
    _writeout(1)


def _make_sc_edge():
    mesh = plsc.VectorSubcoreMesh(core_axis_name="c", subcore_axis_name="s")
    out_type = [
        jax.ShapeDtypeStruct((E, D), jnp.float32),
        jax.ShapeDtypeStruct((4 * ACC, D), jnp.float32),
        jax.ShapeDtypeStruct((2 * NS, D), jnp.float32),
    ]
    idx = pltpu.VMEM((K,), jnp.int32)
    buf = pltpu.VMEM((K, D), jnp.float32)
    scratch = [idx, idx, idx, idx, idx, idx]
    scratch += [buf, buf, buf, buf, buf, buf, buf]
    scratch += [
        pltpu.VMEM((1, D), jnp.float32),
        pltpu.VMEM_SHARED((ACC, D), jnp.float32),
    ]
    return pl.kernel(
        _sc_edge_body,
        out_type=out_type,
        mesh=mesh,
        scratch_types=scratch,
    )


def _node_mm_body(hh_ref, wa_ref, ba_ref, wb_ref, bb_ref, wd_ref, bd_ref,
                  we_ref, be_ref, ah_ref, tsb_ref, teh_ref):
    hh = hh_ref[...]
    ah_ref[...] = jnp.dot(hh, wa_ref[...],
                          preferred_element_type=jnp.float32) + ba_ref[...]
    bh = jnp.dot(hh, wb_ref[...],
                 preferred_element_type=jnp.float32) + bb_ref[...]
    dh = jnp.dot(hh, wd_ref[...],
                 preferred_element_type=jnp.float32) + bd_ref[...]
    teh_ref[...] = jnp.dot(hh, we_ref[...],
                           preferred_element_type=jnp.float32) + be_ref[...]
    tsb_ref[0] = jnp.concatenate([dh[:, :H], bh[:, :H]], axis=1)
    tsb_ref[1] = jnp.concatenate([dh[:, H:], bh[:, H:]], axis=1)


def _node_mm(hh, wa, ba, wb, bb, wd, bd, we, be):
    full = pl.BlockSpec((N, D), lambda: (0, 0))
    wspec = pl.BlockSpec((D, D), lambda: (0, 0))
    bspec = pl.BlockSpec((1, D), lambda: (0, 0))
    return pl.pallas_call(
        _node_mm_body,
        grid=(),
        in_specs=[full, wspec, bspec, wspec, bspec, wspec, bspec, wspec,
                  bspec],
        out_specs=(full, pl.BlockSpec((2, N, D), lambda: (0, 0, 0)), full),
        out_shape=(
            jax.ShapeDtypeStruct((N, D), jnp.float32),
            jax.ShapeDtypeStruct((2, N, D), jnp.float32),
            jax.ShapeDtypeStruct((N, D), jnp.float32),
        ),
    )(hh, wa, ba.reshape(1, D), wb, bb.reshape(1, D), wd, bd.reshape(1, D),
      we, be.reshape(1, D))


def _node_update_body(readout, ah_ref, nd_ref, g_ref, b_ref,
                      hin_ref, *rest):
    ndc = []
    for cc in range(2):
        nd = jnp.concatenate([nd_ref[cc, 0, :RNG, :],
                              nd_ref[cc, 1, :N - RNG, :]], axis=0)
        ndc.append(nd)
    num = jnp.concatenate([ndc[0][:, :H], ndc[1][:, :H]], axis=1)
    den = jnp.concatenate([ndc[0][:, H:], ndc[1][:, H:]], axis=1)
    h_new = ah_ref[...] + num / (den + 1e-6)
    mu = jnp.mean(h_new, axis=0, keepdims=True)
    var = jnp.mean(h_new * h_new, axis=0, keepdims=True) - mu * mu
    h_new = (h_new - mu) * jax.lax.rsqrt(var + 1e-5) * g_ref[...] + b_ref[...]
    hh = jnp.maximum(h_new, 0.0) + hin_ref[...]
    if not readout:
        rest[0][...] = hh
    else:
        (wm0_ref, bm0_ref, wm1_ref, bm1_ref, wm2_ref, bm2_ref,
         y_ref) = rest
        hg = jnp.mean(hh, axis=0, keepdims=True)
        y = jnp.maximum(
            jnp.dot(hg, wm0_ref[...], preferred_element_type=jnp.float32)
            + bm0_ref[...], 0.0)
        y = jnp.maximum(
            jnp.dot(y, wm1_ref[...], preferred_element_type=jnp.float32)
            + bm1_ref[...], 0.0)
        y_ref[...] = (jnp.dot(y, wm2_ref[...],
                              preferred_element_type=jnp.float32)
                      + bm2_ref[...])


def _node_update(ah, nd2, g, b, hin):
    full = pl.BlockSpec((N, D), lambda: (0, 0))
    half = pl.BlockSpec((2, 2, ACC, D), lambda: (0, 0, 0, 0))
    bspec = pl.BlockSpec((1, D), lambda: (0, 0))
    return pl.pallas_call(
        functools.partial(_node_update_body, False),
        grid=(),
        in_specs=[full, half, bspec, bspec, full],
        out_specs=full,
        out_shape=jax.ShapeDtypeStruct((N, D), jnp.float32),
    )(ah, nd2, g.reshape(1, D), b.reshape(1, D), hin)


def _node_update_readout(ah, nd2, g, b, hin, wm0, bm0, wm1, bm1, wm2, bm2):
    full = pl.BlockSpec((N, D), lambda: (0, 0))
    half = pl.BlockSpec((2, 2, ACC, D), lambda: (0, 0, 0, 0))
    bspec = pl.BlockSpec((1, D), lambda: (0, 0))
    return pl.pallas_call(
        functools.partial(_node_update_body, True),
        grid=(),
        in_specs=[full, half, bspec, bspec, full,
                  pl.BlockSpec((D, H), lambda: (0, 0)),
                  pl.BlockSpec((1, H), lambda: (0, 0)),
                  pl.BlockSpec((H, D // 4), lambda: (0, 0)),
                  pl.BlockSpec((1, D // 4), lambda: (0, 0)),
                  pl.BlockSpec((D // 4, 1), lambda: (0, 0)),
                  pl.BlockSpec((1, 1), lambda: (0, 0))],
        out_specs=pl.BlockSpec((1, 1), lambda: (0, 0)),
        out_shape=jax.ShapeDtypeStruct((1, 1), jnp.float32),
    )(ah, nd2, g.reshape(1, D), b.reshape(1, D), hin,
      wm0, bm0.reshape(1, H), wm1, bm1.reshape(1, D // 4), wm2,
      bm2.reshape(1, 1))


def _edge_prep_body(mode, epk_ref, ea_ref, eb_ref, bemb_ref, mu_ref,
                    rs_ref, g_ref, b_ref, wc_ref, bc_ref, *outs):
    enew_a = jnp.concatenate([epk_ref[0, :, :H], epk_ref[1, :, :H]], axis=1)
    enew_b = jnp.concatenate([epk_ref[0, :, H:], epk_ref[1, :, H:]], axis=1)
    rs = rs_ref[...]
    g = g_ref[...]
    b = b_ref[...]
    mu = mu_ref[...]
    ce_ref = outs[-1]
    ces = []
    for half, enew in ((0, enew_a), (1, enew_b)):
        ebn = (enew - mu) * rs * g + b
        if mode == "first":
            ev = (ea_ref if half == 0 else eb_ref)[0, 0, :]
            oh = (ev[:, None] == lax.broadcasted_iota(jnp.int32, (1, 16), 1)
                  ).astype(jnp.float32)
            eprev = jnp.dot(oh, bemb_ref[...],
                            preferred_element_type=jnp.float32)
        else:
            eprev = (ea_ref if half == 0 else eb_ref)[...]
        ee = jnp.maximum(ebn, 0.0) + eprev
        ces.append(jnp.dot(ee, wc_ref[...],
                           preferred_element_type=jnp.float32) + bc_ref[...])
        if mode != "last":
            outs[0][half] = ee
    ce_ref[0] = jnp.concatenate([ces[0][:, :H], ces[1][:, :H]], axis=1)
    ce_ref[1] = jnp.concatenate([ces[0][:, H:], ces[1][:, H:]], axis=1)


def _edge_prep(mode, enewf, eprev, bemb, mu, rstd, g, b, wc, bc):
    full_a = pl.BlockSpec((EBH, D), lambda i: (i, 0))
    full_b = pl.BlockSpec((EBH, D), lambda i: (i + G2, 0))
    bspec = pl.BlockSpec((1, D), lambda i: (0, 0))
    epk_spec = pl.BlockSpec((2, EBH, D), lambda i: (0, i, 0))
    if mode == "first":
        ea_spec = pl.BlockSpec((1, 1, EBH), lambda i: (i, 0, 0))
        eb_spec = pl.BlockSpec((1, 1, EBH), lambda i: (i + G2, 0, 0))
        eprev = eprev.reshape(E // EBH, 1, EBH)
    else:
        ea_spec, eb_spec = full_a, full_b
    out_specs = [pl.BlockSpec((2, EBH, D), lambda i: (0, i, 0))]
    out_shape = [jax.ShapeDtypeStruct((2, EH, D), jnp.float32)]
    if mode != "last":
        out_specs = [pl.BlockSpec((2, EBH, D), lambda i: (0, i, 0))] \
            + out_specs
        out_shape = [jax.ShapeDtypeStruct((2, EH, D), jnp.float32)] \
            + out_shape
    res = pl.pallas_call(
        functools.partial(_edge_prep_body, mode),
        grid=(G2,),
        in_specs=[epk_spec, ea_spec, eb_spec,
                  pl.BlockSpec((16, D), lambda i: (0, 0)),
                  bspec, bspec, bspec, bspec,
                  pl.BlockSpec((D, D), lambda i: (0, 0)), bspec],
        out_specs=tuple(out_specs),
        out_shape=tuple(out_shape),
    )(enewf.reshape(2, EH, D), eprev, eprev, bemb, mu.reshape(1, D),
      rstd.reshape(1, D), g.reshape(1, D), b.reshape(1, D), wc,
      bc.reshape(1, D))
    if mode == "last":
        return None, res[0].reshape(E, D)
    return res[0].reshape(E, D), res[1].reshape(E, D)


def _ce0_body(ea_ref, eb_ref, tab_ref, ce_ref):
    ces = []
    for ref in (ea_ref, eb_ref):
        ev = ref[0, 0, :]
        oh = (ev[:, None] == lax.broadcasted_iota(jnp.int32, (1, 16), 1)
              ).astype(jnp.float32)
        ces.append(jnp.dot(oh, tab_ref[...],
                           preferred_element_type=jnp.float32))
    ce_ref[0] = jnp.concatenate([ces[0][:, :H], ces[1][:, :H]], axis=1)
    ce_ref[1] = jnp.concatenate([ces[0][:, H:], ces[1][:, H:]], axis=1)


def _ce0(e32, tab):
    e3 = e32.reshape(E // EBH, 1, EBH)
    res = pl.pallas_call(
        _ce0_body,
        grid=(G2,),
        in_specs=[pl.BlockSpec((1, 1, EBH), lambda i: (i, 0, 0)),
                  pl.BlockSpec((1, 1, EBH), lambda i: (i + G2, 0, 0)),
                  pl.BlockSpec((16, D), lambda i: (0, 0))],
        out_specs=pl.BlockSpec((2, EBH, D), lambda i: (0, i, 0)),
        out_shape=jax.ShapeDtypeStruct((2, EH, D), jnp.float32),
    )(e3, e3, tab)
    return res.reshape(E, D)


def _stats_to_mu_rstd(stats):
    st = stats.reshape(2, NS, D).sum(axis=1)
    ssum = jnp.concatenate([st[0, :H], st[1, :H]])
    ssq = jnp.concatenate([st[0, H:], st[1, H:]])
    mu = ssum / E
    var = ssq / E - mu * mu
    return mu, lax.rsqrt(var + 1e-5)


def kernel(h, e, edge_index, p, snorm_n, atom_emb, bond_emb, Wp, bp,
           Wa, ba, Wb, bb, Wc, bc, Wd, bd, We, be,
           bn_gh, bn_bh, bn_ge, bn_be, Wm0, bm0, Wm1, bm1, Wm2, bm2):
    sc_edge = _make_sc_edge()

    hh = atom_emb[h] + p @ Wp + bp
    src = edge_index[0].astype(jnp.int32)
    dst = edge_index[1].astype(jnp.int32)
    e32 = e.astype(jnp.int32)

    cetab = bond_emb @ Wc[0] + bc[0]

    y = None
    ee = None
    enewf = None
    stats = None
    for l in range(L):
        if l > 0:
            mu_e, rstd_e = _stats_to_mu_rstd(stats)
            mode = "first" if l == 1 else ("last" if l == L - 1 else "mid")
            eprev = e32 if l == 1 else ee
            ee, cef = _edge_prep(mode, enewf, eprev, bond_emb, mu_e,
                                 rstd_e, bn_ge[l - 1], bn_be[l - 1],
                                 Wc[l], bc[l])
        ah, tsb, teh = _node_mm(hh, Wa[l], ba[l], Wb[l], bb[l],
                                Wd[l], bd[l], We[l], be[l])
        tsb = tsb.reshape(2 * N, D)
        if l == 0:
            cef = _ce0(e32, cetab)
        enewf, nd2, stats = sc_edge(src, dst, tsb, teh, cef)
        nd2 = nd2.reshape(2, 2, ACC, D)
        if l < L - 1:
            hh = _node_update(ah, nd2, bn_gh[l], bn_bh[l], hh)
        else:
            y = _node_update_readout(ah, nd2, bn_gh[l], bn_bh[l], hh,
                                     Wm0, bm0, Wm1, bm1, Wm2, bm2)
    return y.reshape(1)


# speedup vs baseline: 1.1571x; 1.0655x over previous
import functools

import jax
import jax.numpy as jnp
from jax import lax
from jax.experimental import pallas as pl
from jax.experimental.pallas import tpu as pltpu
from jax.experimental.pallas import tpu_sc as plsc

N = 10000
E = 320000
EH = E // 2
D = 128
L = 4
H = 64
NPAD = 10240
NC = 2
NS = 16
RPT = EH // NS
K = 80
CH = RPT // K
NPT = NPAD // NS

RNG = 5056
ACC = 5120
APT = ACC // NS

EBH = 1600
G2 = EH // EBH


def _sc_edge_body(*refs):
    (src_hbm, dst_hbm, tsb, teh, ce_hbm,
     enew_hbm, nd_hbm, stats_hbm, pay0_hbm, pay1_hbm,
     s0v, s1v, d0v, d1v, d0x, d1x,
     bufT0, bufT1, bufE0, bufE1, bufC, bufM0, bufM1,
     statb, acc) = refs
    c = lax.axis_index("c")
    s = lax.axis_index("s")
    wid = c * NS + s

    zero = jnp.zeros((16,), jnp.float32)

    def _zero_acc():
        def _zrow(r, _):
            for j in range(D // 16):
                bufM0[r, pl.ds(j * 16, 16)] = zero
            return 0
        lax.fori_loop(0, K, _zrow, 0)
        for k in range(APT // K):
            pltpu.sync_copy(bufM0, acc.at[pl.ds(s * APT + k * K, K)])
        plsc.subcore_barrier()

    def _clamp(dv, xv, lo):
        for j in range(K // 16):
            sl = pl.ds(j * 16, 16)
            d = dv[sl] - lo
            ok = (d >= 0) & (d < RNG)
            xv[sl] = jnp.where(ok, d, RNG)

    def _writeout(p):
        plsc.subcore_barrier()
        for k in range(APT // K):
            nsl = pl.ds(s * APT + k * K, K)
            osl = pl.ds(c * (2 * ACC) + p * ACC + s * APT + k * K, K)
            pltpu.sync_copy(acc.at[nsl], nd_hbm.at[osl])
        plsc.subcore_barrier()

    _zero_acc()

    def _chunk(i, carry):
        base = s * RPT + i * K
        pltpu.sync_copy(src_hbm.at[pl.ds(base, K)], s0v)
        pltpu.sync_copy(src_hbm.at[pl.ds(EH + base, K)], s1v)
        pltpu.sync_copy(dst_hbm.at[pl.ds(base, K)], d0v)
        pltpu.sync_copy(dst_hbm.at[pl.ds(EH + base, K)], d1v)
        coff = c * N
        for j in range(K // 16):
            sl = pl.ds(j * 16, 16)
            s0v[sl] = s0v[sl] + coff
            s1v[sl] = s1v[sl] + coff
        pltpu.sync_copy(tsb.at[s0v], bufT0)
        pltpu.sync_copy(tsb.at[s1v], bufT1)
        pltpu.sync_copy(teh.at[d0v], bufE0)
        pltpu.sync_copy(teh.at[d1v], bufE1)
        pltpu.sync_copy(ce_hbm.at[pl.ds(c * EH + base, K)], bufC)

        def _row(q, cr):
            acc_c = list(cr)
            for half in range(2):
                bt = bufT0 if half == 0 else bufT1
                be = bufE0 if half == 0 else bufE1
                bm = bufM0 if half == 0 else bufM1
                for j in range(H // 16):
                    slj = pl.ds(j * 16, 16)
                    slc = pl.ds(half * H + j * 16, 16)
                    sle = pl.ds(c * H + j * 16, 16)
                    slb = pl.ds(H + j * 16, 16)
                    x = bt[q, slj] + be[q, sle] + bufC[q, slc]
                    sg = 1.0 / (1.0 + jnp.exp(-x))
                    bufC[q, slc] = x
                    bm[q, slj] = bt[q, slb] * sg
                    bm[q, slb] = sg
                    acc_c[j] = acc_c[j] + x
                    acc_c[4 + j] = acc_c[4 + j] + x * x
            return tuple(acc_c)

        carry = lax.fori_loop(0, K, _row, carry)
        bb = c * EH + base
        pltpu.sync_copy(bufC, enew_hbm.at[pl.ds(bb, K)])
        pltpu.sync_copy(bufM0, pay0_hbm.at[pl.ds(bb, K)])
        pltpu.sync_copy(bufM1, pay1_hbm.at[pl.ds(bb, K)])
        _clamp(d0v, d0x, 0)
        _clamp(d1v, d1x, 0)
        pltpu.sync_copy(bufM0, acc.at[d0x], add=True)
        pltpu.sync_copy(bufM1, acc.at[d1x], add=True)
        return carry

    init = tuple(zero for _ in range(8))
    carry = lax.fori_loop(0, CH, _chunk, init)

    for j in range(H // 16):
        statb[0, pl.ds(j * 16, 16)] = carry[j]
        statb[0, pl.ds(H + j * 16, 16)] = carry[4 + j]
    pltpu.sync_copy(statb, stats_hbm.at[pl.ds(wid, 1)])

    _writeout(0)

    _zero_acc()

    def _chunk2(i, _):
        base = s * RPT + i * K
        pltpu.sync_copy(dst_hbm.at[pl.ds(base, K)], d0v)
        pltpu.sync_copy(dst_hbm.at[pl.ds(EH + base, K)], d1v)
        _clamp(d0v, d0x, RNG)
        _clamp(d1v, d1x, RNG)
        bb = c * EH + base
        pltpu.sync_copy(pay0_hbm.at[pl.ds(bb, K)], bufM0)
        pltpu.sync_copy(pay1_hbm.at[pl.ds(bb, K)], bufM1)
        pltpu.sync_copy(bufM0, acc.at[d0x], add=True)
        pltpu.sync_copy(bufM1, acc.at[d1x], add=True)
        return 0

    lax.fori_loop(0, CH, _chunk2, 0)
    _writeout(1)


def _make_sc_edge():
    mesh = plsc.VectorSubcoreMesh(core_axis_name="c", subcore_axis_name="s")
    out_type = [
        jax.ShapeDtypeStruct((E, D), jnp.float32),
        jax.ShapeDtypeStruct((4 * ACC, D), jnp.float32),
        jax.ShapeDtypeStruct((2 * NS, D), jnp.float32),
        jax.ShapeDtypeStruct((E, D), jnp.float32),
        jax.ShapeDtypeStruct((E, D), jnp.float32),
    ]
    idx = pltpu.VMEM((K,), jnp.int32)
    buf = pltpu.VMEM((K, D), jnp.float32)
    scratch = [idx, idx, idx, idx, idx, idx]
    scratch += [buf, buf, buf, buf, buf, buf, buf]
    scratch += [
        pltpu.VMEM((1, D), jnp.float32),
        pltpu.VMEM_SHARED((ACC, D), jnp.float32),
    ]
    return pl.kernel(
        _sc_edge_body,
        out_type=out_type,
        mesh=mesh,
        scratch_types=scratch,
    )


def _node_mm_body(hh_ref, wa_ref, ba_ref, wb_ref, bb_ref, wd_ref, bd_ref,
                  we_ref, be_ref, ah_ref, tsb_ref, teh_ref):
    hh = hh_ref[...]
    ah_ref[...] = jnp.dot(hh, wa_ref[...],
                          preferred_element_type=jnp.float32) + ba_ref[...]
    bh = jnp.dot(hh, wb_ref[...],
                 preferred_element_type=jnp.float32) + bb_ref[...]
    dh = jnp.dot(hh, wd_ref[...],
                 preferred_element_type=jnp.float32) + bd_ref[...]
    teh_ref[...] = jnp.dot(hh, we_ref[...],
                           preferred_element_type=jnp.float32) + be_ref[...]
    tsb_ref[0] = jnp.concatenate([dh[:, :H], bh[:, :H]], axis=1)
    tsb_ref[1] = jnp.concatenate([dh[:, H:], bh[:, H:]], axis=1)


def _node_mm(hh, wa, ba, wb, bb, wd, bd, we, be):
    full = pl.BlockSpec((N, D), lambda: (0, 0))
    wspec = pl.BlockSpec((D, D), lambda: (0, 0))
    bspec = pl.BlockSpec((1, D), lambda: (0, 0))
    return pl.pallas_call(
        _node_mm_body,
        grid=(),
        in_specs=[full, wspec, bspec, wspec, bspec, wspec, bspec, wspec,
                  bspec],
        out_specs=(full, pl.BlockSpec((2, N, D), lambda: (0, 0, 0)), full),
        out_shape=(
            jax.ShapeDtypeStruct((N, D), jnp.float32),
            jax.ShapeDtypeStruct((2, N, D), jnp.float32),
            jax.ShapeDtypeStruct((N, D), jnp.float32),
        ),
    )(hh, wa, ba.reshape(1, D), wb, bb.reshape(1, D), wd, bd.reshape(1, D),
      we, be.reshape(1, D))


def _node_update_body(readout, ah_ref, nd_ref, g_ref, b_ref,
                      hin_ref, *rest):
    ndc = []
    for cc in range(2):
        nd = jnp.concatenate([nd_ref[cc, 0, :RNG, :],
                              nd_ref[cc, 1, :N - RNG, :]], axis=0)
        ndc.append(nd)
    num = jnp.concatenate([ndc[0][:, :H], ndc[1][:, :H]], axis=1)
    den = jnp.concatenate([ndc[0][:, H:], ndc[1][:, H:]], axis=1)
    h_new = ah_ref[...] + num / (den + 1e-6)
    mu = jnp.mean(h_new, axis=0, keepdims=True)
    var = jnp.mean(h_new * h_new, axis=0, keepdims=True) - mu * mu
    h_new = (h_new - mu) * jax.lax.rsqrt(var + 1e-5) * g_ref[...] + b_ref[...]
    hh = jnp.maximum(h_new, 0.0) + hin_ref[...]
    if not readout:
        rest[0][...] = hh
    else:
        (wm0_ref, bm0_ref, wm1_ref, bm1_ref, wm2_ref, bm2_ref,
         y_ref) = rest
        hg = jnp.mean(hh, axis=0, keepdims=True)
        y = jnp.maximum(
            jnp.dot(hg, wm0_ref[...], preferred_element_type=jnp.float32)
            + bm0_ref[...], 0.0)
        y = jnp.maximum(
            jnp.dot(y, wm1_ref[...], preferred_element_type=jnp.float32)
            + bm1_ref[...], 0.0)
        y_ref[...] = (jnp.dot(y, wm2_ref[...],
                              preferred_element_type=jnp.float32)
                      + bm2_ref[...])


def _node_update(ah, nd2, g, b, hin):
    full = pl.BlockSpec((N, D), lambda: (0, 0))
    half = pl.BlockSpec((2, 2, ACC, D), lambda: (0, 0, 0, 0))
    bspec = pl.BlockSpec((1, D), lambda: (0, 0))
    return pl.pallas_call(
        functools.partial(_node_update_body, False),
        grid=(),
        in_specs=[full, half, bspec, bspec, full],
        out_specs=full,
        out_shape=jax.ShapeDtypeStruct((N, D), jnp.float32),
    )(ah, nd2, g.reshape(1, D), b.reshape(1, D), hin)


def _node_update_readout(ah, nd2, g, b, hin, wm0, bm0, wm1, bm1, wm2, bm2):
    full = pl.BlockSpec((N, D), lambda: (0, 0))
    half = pl.BlockSpec((2, 2, ACC, D), lambda: (0, 0, 0, 0))
    bspec = pl.BlockSpec((1, D), lambda: (0, 0))
    return pl.pallas_call(
        functools.partial(_node_update_body, True),
        grid=(),
        in_specs=[full, half, bspec, bspec, full,
                  pl.BlockSpec((D, H), lambda: (0, 0)),
                  pl.BlockSpec((1, H), lambda: (0, 0)),
                  pl.BlockSpec((H, D // 4), lambda: (0, 0)),
                  pl.BlockSpec((1, D // 4), lambda: (0, 0)),
                  pl.BlockSpec((D // 4, 1), lambda: (0, 0)),
                  pl.BlockSpec((1, 1), lambda: (0, 0))],
        out_specs=pl.BlockSpec((1, 1), lambda: (0, 0)),
        out_shape=jax.ShapeDtypeStruct((1, 1), jnp.float32),
    )(ah, nd2, g.reshape(1, D), b.reshape(1, D), hin,
      wm0, bm0.reshape(1, H), wm1, bm1.reshape(1, D // 4), wm2,
      bm2.reshape(1, 1))


def _edge_prep_body(mode, epk_ref, ea_ref, eb_ref, bemb_ref, mu_ref,
                    rs_ref, g_ref, b_ref, wc_ref, bc_ref, *outs):
    enew_a = jnp.concatenate([epk_ref[0, :, :H], epk_ref[1, :, :H]], axis=1)
    enew_b = jnp.concatenate([epk_ref[0, :, H:], epk_ref[1, :, H:]], axis=1)
    rs = rs_ref[...]
    g = g_ref[...]
    b = b_ref[...]
    mu = mu_ref[...]
    ce_ref = outs[-1]
    ces = []
    for half, enew in ((0, enew_a), (1, enew_b)):
        ebn = (enew - mu) * rs * g + b
        if mode == "first":
            ev = (ea_ref if half == 0 else eb_ref)[0, 0, :]
            oh = (ev[:, None] == lax.broadcasted_iota(jnp.int32, (1, 16), 1)
                  ).astype(jnp.float32)
            eprev = jnp.dot(oh, bemb_ref[...],
                            preferred_element_type=jnp.float32)
        else:
            eprev = (ea_ref if half == 0 else eb_ref)[...]
        ee = jnp.maximum(ebn, 0.0) + eprev
        ces.append(jnp.dot(ee, wc_ref[...],
                           preferred_element_type=jnp.float32) + bc_ref[...])
        if mode != "last":
            outs[0][half] = ee
    ce_ref[0] = jnp.concatenate([ces[0][:, :H], ces[1][:, :H]], axis=1)
    ce_ref[1] = jnp.concatenate([ces[0][:, H:], ces[1][:, H:]], axis=1)


def _edge_prep(mode, enewf, eprev, bemb, mu, rstd, g, b, wc, bc):
    full_a = pl.BlockSpec((EBH, D), lambda i: (i, 0))
    full_b = pl.BlockSpec((EBH, D), lambda i: (i + G2, 0))
    bspec = pl.BlockSpec((1, D), lambda i: (0, 0))
    epk_spec = pl.BlockSpec((2, EBH, D), lambda i: (0, i, 0))
    if mode == "first":
        ea_spec = pl.BlockSpec((1, 1, EBH), lambda i: (i, 0, 0))
        eb_spec = pl.BlockSpec((1, 1, EBH), lambda i: (i + G2, 0, 0))
        eprev = eprev.reshape(E // EBH, 1, EBH)
    else:
        ea_spec, eb_spec = full_a, full_b
    out_specs = [pl.BlockSpec((2, EBH, D), lambda i: (0, i, 0))]
    out_shape = [jax.ShapeDtypeStruct((2, EH, D), jnp.float32)]
    if mode != "last":
        out_specs = [pl.BlockSpec((2, EBH, D), lambda i: (0, i, 0))] \
            + out_specs
        out_shape = [jax.ShapeDtypeStruct((2, EH, D), jnp.float32)] \
            + out_shape
    res = pl.pallas_call(
        functools.partial(_edge_prep_body, mode),
        grid=(G2,),
        in_specs=[epk_spec, ea_spec, eb_spec,
                  pl.BlockSpec((16, D), lambda i: (0, 0)),
                  bspec, bspec, bspec, bspec,
                  pl.BlockSpec((D, D), lambda i: (0, 0)), bspec],
        out_specs=tuple(out_specs),
        out_shape=tuple(out_shape),
    )(enewf.reshape(2, EH, D), eprev, eprev, bemb, mu.reshape(1, D),
      rstd.reshape(1, D), g.reshape(1, D), b.reshape(1, D), wc,
      bc.reshape(1, D))
    if mode == "last":
        return None, res[0].reshape(E, D)
    return res[0].reshape(E, D), res[1].reshape(E, D)


def _ce0_body(ea_ref, eb_ref, tab_ref, ce_ref):
    ces = []
    for ref in (ea_ref, eb_ref):
        ev = ref[0, 0, :]
        oh = (ev[:, None] == lax.broadcasted_iota(jnp.int32, (1, 16), 1)
              ).astype(jnp.float32)
        ces.append(jnp.dot(oh, tab_ref[...],
                           preferred_element_type=jnp.float32))
    ce_ref[0] = jnp.concatenate([ces[0][:, :H], ces[1][:, :H]], axis=1)
    ce_ref[1] = jnp.concatenate([ces[0][:, H:], ces[1][:, H:]], axis=1)


def _ce0(e32, tab):
    e3 = e32.reshape(E // EBH, 1, EBH)
    res = pl.pallas_call(
        _ce0_body,
        grid=(G2,),
        in_specs=[pl.BlockSpec((1, 1, EBH), lambda i: (i, 0, 0)),
                  pl.BlockSpec((1, 1, EBH), lambda i: (i + G2, 0, 0)),
                  pl.BlockSpec((16, D), lambda i: (0, 0))],
        out_specs=pl.BlockSpec((2, EBH, D), lambda i: (0, i, 0)),
        out_shape=jax.ShapeDtypeStruct((2, EH, D), jnp.float32),
    )(e3, e3, tab)
    return res.reshape(E, D)


def _stats_to_mu_rstd(stats):
    st = stats.reshape(2, NS, D).sum(axis=1)
    ssum = jnp.concatenate([st[0, :H], st[1, :H]])
    ssq = jnp.concatenate([st[0, H:], st[1, H:]])
    mu = ssum / E
    var = ssq / E - mu * mu
    return mu, lax.rsqrt(var + 1e-5)


def kernel(h, e, edge_index, p, snorm_n, atom_emb, bond_emb, Wp, bp,
           Wa, ba, Wb, bb, Wc, bc, Wd, bd, We, be,
           bn_gh, bn_bh, bn_ge, bn_be, Wm0, bm0, Wm1, bm1, Wm2, bm2):
    sc_edge = _make_sc_edge()

    hh = atom_emb[h] + p @ Wp + bp
    src = edge_index[0].astype(jnp.int32)
    dst = edge_index[1].astype(jnp.int32)
    e32 = e.astype(jnp.int32)

    cetab = bond_emb @ Wc[0] + bc[0]

    y = None
    ee = None
    enewf = None
    stats = None
    for l in range(L):
        if l > 0:
            mu_e, rstd_e = _stats_to_mu_rstd(stats)
            mode = "first" if l == 1 else ("last" if l == L - 1 else "mid")
            eprev = e32 if l == 1 else ee
            ee, cef = _edge_prep(mode, enewf, eprev, bond_emb, mu_e,
                                 rstd_e, bn_ge[l - 1], bn_be[l - 1],
                                 Wc[l], bc[l])
        ah, tsb, teh = _node_mm(hh, Wa[l], ba[l], Wb[l], bb[l],
                                Wd[l], bd[l], We[l], be[l])
        tsb = tsb.reshape(2 * N, D)
        if l == 0:
            cef = _ce0(e32, cetab)
        enewf, nd2, stats, _, _ = sc_edge(src, dst, tsb, teh, cef)
        nd2 = nd2.reshape(2, 2, ACC, D)
        if l < L - 1:
            hh = _node_update(ah, nd2, bn_gh[l], bn_bh[l], hh)
        else:
            y = _node_update_readout(ah, nd2, bn_gh[l], bn_bh[l], hh,
                                     Wm0, bm0, Wm1, bm1, Wm2, bm2)
    return y.reshape(1)
